# baseline (speedup 1.0000x reference)
import numpy as np
import ml_dtypes
import jax
import jax.numpy as jnp
from jax.experimental import pallas as pl
from jax.experimental.pallas import tpu as pltpu

_BB = 512


def _sel(n_out, n_in, k, off):
    r = np.zeros((n_out, n_in, k), np.float32)
    p = np.arange(n_out)[:, None]
    t = np.arange(k)[None, :]
    r[p, 2 * p + off + t, t] = 1.0
    return r


_R1S = np.stack([_sel(12, 28, 5, d) for d in (0, 1)])
_R2S = np.stack([_sel(4, 12, 5, d) for d in (0, 1)])


def _pack_kernel(a1op_ref, a2op_ref, g_ref, h_ref):
    g_ref[...] = jnp.zeros(g_ref.shape, g_ref.dtype)
    h_ref[...] = jnp.zeros(h_ref.shape, h_ref.dtype)
    for d in range(2):
        for p in range(12):
            for kh in range(5):
                r = 2 * p + d + kh
                g_ref[r * 28:(r + 1) * 28,
                      d * 1536 + p * 128:d * 1536 + (p + 1) * 128] = a1op_ref[0, kh]
                pa = p - d - kh
                if pa >= 0 and pa < 8 and pa % 2 == 0:
                    h_ref[p * 128:p * 128 + 120,
                          d * 512 + (pa // 2) * 128:d * 512 + (pa // 2 + 1) * 128] \
                        = a2op_ref[0, kh]


def _net_kernel(x_ref, g_ref, h_ref, b1_ref, b2_ref,
                f1_ref, fb1_ref, f2_ref, fb2_ref, o_ref):
    x = x_ref[...].astype(jnp.bfloat16).reshape(x_ref.shape[0], 784)

    z = jnp.dot(x, g_ref[...], preferred_element_type=jnp.float32).astype(jnp.bfloat16)
    a1 = jnp.maximum(jnp.maximum(z[:, :1536], z[:, 1536:3072]),
                     jnp.maximum(z[:, 3072:4608], z[:, 4608:]))
    a1 = jnp.maximum(a1 + b1_ref[...], 0.0)

    z = jnp.dot(a1, h_ref[...], preferred_element_type=jnp.float32).astype(jnp.bfloat16)
    a2 = jnp.maximum(jnp.maximum(z[:, :512], z[:, 512:1024]),
                     jnp.maximum(z[:, 1024:1536], z[:, 1536:]))
    a2 = jnp.maximum(a2 + b2_ref[...], 0.0)

    h = jnp.dot(a2, f1_ref[...], preferred_element_type=jnp.float32) + fb1_ref[...]
    h = jnp.maximum(h, 0.0).astype(jnp.bfloat16)
    logits = jnp.dot(h, f2_ref[...], preferred_element_type=jnp.float32) + fb2_ref[...]
    m = jnp.max(logits, axis=-1, keepdims=True)
    lse = jnp.log(jnp.sum(jnp.exp(logits - m), axis=-1, keepdims=True)) + m
    o_ref[...] = logits - lse


def kernel(c1_w, c1_b, c2_w, c2_b, fc1_w, fc1_b, fc2_w, fc2_b, x):
    B = x.shape[0]
    xr = x.reshape(B, 28, 28)

    w1 = c1_w[:, :10].reshape(5, 5, 10)
    a1op = (jnp.einsum('eqsw,hwc->ehsqc', _R1S, w1)
            .reshape(2, 5, 28, 120).astype(jnp.bfloat16))
    a1op = jnp.pad(a1op, ((0, 0), (0, 0), (0, 0), (0, 8)))
    w2 = c2_w[:, :20].reshape(5, 5, 10, 20)
    a2op = (jnp.einsum('ebqw,hwic->ehqibc', _R2S, w2)
            .reshape(2, 5, 120, 80).astype(jnp.bfloat16))
    a2op = jnp.pad(a2op, ((0, 0), (0, 0), (0, 0), (0, 48)))

    g_all, h_all = pl.pallas_call(
        _pack_kernel,
        out_shape=(jax.ShapeDtypeStruct((784, 6144), jnp.bfloat16),
                   jax.ShapeDtypeStruct((1536, 2048), jnp.bfloat16)),
        grid=(2,),
        in_specs=[pl.BlockSpec((1, 5, 28, 128), lambda e: (e, 0, 0, 0)),
                  pl.BlockSpec((1, 5, 120, 128), lambda e: (e, 0, 0, 0))],
        out_specs=(pl.BlockSpec((784, 3072), lambda e: (0, e)),
                   pl.BlockSpec((1536, 1024), lambda e: (0, e))),
        compiler_params=pltpu.CompilerParams(dimension_semantics=("parallel",)),
    )(a1op, a2op)

    b1l = jnp.pad(jnp.tile(c1_b[0, :10], 12).reshape(1, 120), ((0, 0), (0, 8)))
    b1l = jnp.tile(b1l, (1, 12)).astype(jnp.bfloat16)
    b2l = jnp.pad(jnp.tile(c2_b[0, :20], 4).reshape(1, 80), ((0, 0), (0, 48)))
    b2l = jnp.tile(b2l, (1, 4)).astype(jnp.bfloat16)
    f1 = jnp.pad(fc1_w.reshape(4, 80, 128), ((0, 0), (0, 48), (0, 0)))
    f1 = f1.reshape(512, 128).astype(jnp.bfloat16)
    f2 = fc2_w.astype(jnp.bfloat16)

    const = lambda shape: pl.BlockSpec(shape, lambda i: tuple(0 for _ in shape))
    out = pl.pallas_call(
        _net_kernel,
        out_shape=jax.ShapeDtypeStruct((B, 128), jnp.float32),
        grid=(B // _BB,),
        in_specs=[pl.BlockSpec((_BB, 28, 28), lambda i: (i, 0, 0)),
                  const((784, 6144)), const((1536, 2048)),
                  const((1, 1536)), const((1, 512)),
                  const((512, 128)), const((1, 128)),
                  const((128, 128)), const((1, 128))],
        out_specs=pl.BlockSpec((_BB, 128), lambda i: (i, 0)),
        compiler_params=pltpu.CompilerParams(dimension_semantics=("parallel",)),
    )(xr, g_all, h_all, b1l, b2l, f1, fc1_b, f2, fc2_b)
    return out[:, :10]
